# baseline (speedup 1.0000x reference)
import functools

import jax
import jax.numpy as jnp
from jax import lax
from jax.experimental import pallas as pl
from jax.experimental.pallas import tpu as pltpu
from jax.experimental.pallas import tpu_sc as plsc

N_NODES = 10000
N_EDGES = 320000
IN_CH = 128
HID = 64
OUT_CH = 128
HALF = HID // 2

_INFO = plsc.get_sparse_core_info()
NC = _INFO.num_cores
NS = _INFO.num_subcores
NW = NC * NS
EPW = N_EDGES // NW
K = 200
NCH = EPW // K
NBUF = 5
NRND = (NCH - 1) // NBUF
NPAD = 10240
SPT = NPAD // NS
DEG_C = 8

_SC_PARAMS = pltpu.CompilerParams(use_tc_tiling_on_sc=False)


def _sc_mesh():
    return plsc.VectorSubcoreMesh(core_axis_name="c", subcore_axis_name="s")


def _make_deg_kernel():
    @functools.partial(
        pl.kernel,
        mesh=_sc_mesh(),
        compiler_params=_SC_PARAMS,
        out_type=jax.ShapeDtypeStruct((NC, NPAD, DEG_C), jnp.float32),
        scratch_types=[
            pltpu.VMEM((EPW,), jnp.int32),
            pltpu.VMEM((K, DEG_C), jnp.float32),
            pltpu.VMEM_SHARED((NPAD, DEG_C), jnp.float32),
            pltpu.SemaphoreType.DMA,
        ],
    )
    def deg_kernel(dst_hbm, ones_hbm, zeros_hbm, out_hbm, dst_v, ones_v, acc_sh, sem):
        cid = lax.axis_index("c")
        sid = lax.axis_index("s")
        wid = sid * NC + cid

        pltpu.sync_copy(zeros_hbm, acc_sh.at[pl.ds(sid * SPT, SPT)])
        pltpu.sync_copy(dst_hbm.at[pl.ds(wid * EPW, EPW)], dst_v)
        pltpu.sync_copy(ones_hbm, ones_v)
        plsc.subcore_barrier()

        def body(j, carry):
            for t in range(8):
                i = 8 * j + t
                pltpu.async_copy(
                    ones_v, acc_sh.at[dst_v.at[pl.ds(i * K, K)]], sem, add=True
                )
            for t in range(8):
                pltpu.make_async_copy(
                    ones_v, acc_sh.at[dst_v.at[pl.ds(0, K)]], sem
                ).wait()
            return carry

        lax.fori_loop(0, NCH // 8, body, 0)
        for t in range(NCH % 8):
            i = (NCH // 8) * 8 + t
            pltpu.async_copy(
                ones_v, acc_sh.at[dst_v.at[pl.ds(i * K, K)]], sem, add=True
            )
        for t in range(NCH % 8):
            pltpu.make_async_copy(ones_v, acc_sh.at[dst_v.at[pl.ds(0, K)]], sem).wait()

        plsc.subcore_barrier()
        pltpu.sync_copy(
            acc_sh.at[pl.ds(sid * SPT, SPT)],
            out_hbm.at[cid, pl.ds(sid * SPT, SPT)],
        )

    return deg_kernel


def _make_agg_kernel(C):
    @functools.partial(
        pl.kernel,
        mesh=_sc_mesh(),
        compiler_params=_SC_PARAMS,
        out_type=jax.ShapeDtypeStruct((NC, NPAD, C), jnp.float32),
        scratch_types=[
            pltpu.VMEM((EPW,), jnp.int32),
            pltpu.VMEM((EPW,), jnp.int32),
            [pltpu.VMEM((K, C), jnp.float32)] * NBUF,
            [pltpu.SemaphoreType.DMA] * NBUF,
            [pltpu.SemaphoreType.DMA] * NBUF,
            pltpu.VMEM_SHARED((NPAD, C), jnp.float32),
        ],
    )
    def agg_kernel(src_hbm, dst_hbm, g_hbm, zeros_hbm, out_hbm,
                   src_v, dst_v, bufs, gsems, ssems, acc_sh):
        cid = lax.axis_index("c")
        sid = lax.axis_index("s")
        wid = sid * NC + cid

        pltpu.sync_copy(zeros_hbm, acc_sh.at[pl.ds(sid * SPT, SPT)])
        pltpu.sync_copy(src_hbm.at[pl.ds(wid * EPW, EPW)], src_v)
        pltpu.sync_copy(dst_hbm.at[pl.ds(wid * EPW, EPW)], dst_v)
        plsc.subcore_barrier()

        def gather(i, b):
            return pltpu.async_copy(
                g_hbm.at[src_v.at[pl.ds(i * K, K)]], bufs[b], gsems[b]
            )

        def scatter(i, b):
            return pltpu.async_copy(
                bufs[b], acc_sh.at[dst_v.at[pl.ds(i * K, K)]], ssems[b], add=True
            )

        def wait_gather(b):
            pltpu.make_async_copy(g_hbm.at[src_v.at[pl.ds(0, K)]], bufs[b],
                                  gsems[b]).wait()

        def wait_scatter(b):
            pltpu.make_async_copy(bufs[b], acc_sh.at[dst_v.at[pl.ds(0, K)]],
                                  ssems[b]).wait()

        for b in range(NBUF):
            gather(b, b)

        def body(j, carry):
            i0 = NBUF * j
            for b in range(NBUF):
                wait_gather(b)
                scatter(i0 + b, b)
            for b in range(NBUF):
                wait_scatter(b)
                nxt = i0 + NBUF + b

                @pl.when(nxt < NCH)
                def _():
                    gather(nxt, b)

            return carry

        lax.fori_loop(0, NRND, body, 0)
        for t in range(NCH - NBUF * NRND):
            wait_gather(t)
            scatter(NBUF * NRND + t, t)
        for t in range(NCH - NBUF * NRND):
            wait_scatter(t)

        plsc.subcore_barrier()
        pltpu.sync_copy(
            acc_sh.at[pl.ds(sid * SPT, SPT)],
            out_hbm.at[cid, pl.ds(sid * SPT, SPT)],
        )

    return agg_kernel


_deg_kernel = _make_deg_kernel()
_agg_kernel = _make_agg_kernel(HALF)


_RB = 2000
_GRID = N_NODES // _RB


def _tc1a_body(x_ref, w1_ref, h_ref):
    h_ref[...] = jnp.dot(
        x_ref[...], w1_ref[...], preferred_element_type=jnp.float32
    )


def _tc1b_body(h_ref, d_ref, ga_ref, gb_ref, dis_ref):
    deg = d_ref[0] + d_ref[1] + 1.0
    dis = lax.rsqrt(deg)
    g = h_ref[...] * dis
    ga_ref[...] = g[:, :HALF]
    gb_ref[...] = g[:, HALF:]
    dis_ref[...] = jnp.broadcast_to(dis, (_RB, 8))


def _tc2_body(p_ref, g_ref, dis_ref, b1_ref, u_ref):
    dis = dis_ref[:, 0:1]
    s = p_ref[0] + p_ref[1] + g_ref[...]
    u_ref[...] = dis * jnp.maximum(dis * s + b1_ref[...], 0.0)


def _tc3_body(qa_ref, qb_ref, ua_ref, ub_ref, dis_ref, b2_ref, w2_ref, out_ref):
    dis = dis_ref[:, 0:1]
    sa = dis * (qa_ref[0] + qa_ref[1] + ua_ref[...])
    sb = dis * (qb_ref[0] + qb_ref[1] + ub_ref[...])
    s = jnp.concatenate([sa, sb], axis=1)
    out_ref[...] = (
        jnp.dot(s, w2_ref[...], preferred_element_type=jnp.float32) + b2_ref[...]
    )


def _row_spec(c):
    return pl.BlockSpec((_RB, c), lambda i: (i, 0))


def _part_spec(c):
    return pl.BlockSpec((NC, _RB, c), lambda i: (0, i, 0))


def _full_spec(r, c):
    return pl.BlockSpec((r, c), lambda i: (0, 0))


def _tc2(p, g, dis, b1h):
    return pl.pallas_call(
        _tc2_body,
        grid=(_GRID,),
        in_specs=[
            _part_spec(HALF),
            _row_spec(HALF),
            _row_spec(8),
            _full_spec(1, HALF),
        ],
        out_specs=_row_spec(HALF),
        out_shape=jax.ShapeDtypeStruct((N_NODES, HALF), jnp.float32),
    )(p, g, dis, b1h)


def kernel(x, edge_index, W1, b1, W2, b2):
    ei = edge_index.astype(jnp.int32)
    src = ei[0]
    dst = ei[1]

    ones_deg = jnp.ones((K, DEG_C), jnp.float32)
    zeros_deg = jnp.zeros((SPT, DEG_C), jnp.float32)
    zeros_h = jnp.zeros((SPT, HALF), jnp.float32)

    deg_parts = _deg_kernel(dst, ones_deg, zeros_deg)[:, :, 0:1]

    h1 = pl.pallas_call(
        _tc1a_body,
        grid=(_GRID,),
        in_specs=[_row_spec(IN_CH), _full_spec(IN_CH, HID)],
        out_specs=_row_spec(HID),
        out_shape=jax.ShapeDtypeStruct((N_NODES, HID), jnp.float32),
    )(x, W1)

    g1a, g1b, dis = pl.pallas_call(
        _tc1b_body,
        grid=(_GRID,),
        in_specs=[_row_spec(HID), _part_spec(1)],
        out_specs=[_row_spec(HALF), _row_spec(HALF), _row_spec(8)],
        out_shape=[
            jax.ShapeDtypeStruct((N_NODES, HALF), jnp.float32),
            jax.ShapeDtypeStruct((N_NODES, HALF), jnp.float32),
            jax.ShapeDtypeStruct((N_NODES, 8), jnp.float32),
        ],
    )(h1, deg_parts)

    agg1a = _agg_kernel(src, dst, g1a, zeros_h)
    agg1b = _agg_kernel(src, dst, g1b, zeros_h)
    ua = _tc2(agg1a, g1a, dis, b1[:HALF].reshape(1, HALF))
    agg2a = _agg_kernel(src, dst, ua, zeros_h)
    ub = _tc2(agg1b, g1b, dis, b1[HALF:].reshape(1, HALF))
    agg2b = _agg_kernel(src, dst, ub, zeros_h)

    out = pl.pallas_call(
        _tc3_body,
        grid=(_GRID,),
        in_specs=[
            _part_spec(HALF),
            _part_spec(HALF),
            _row_spec(HALF),
            _row_spec(HALF),
            _row_spec(8),
            _full_spec(1, OUT_CH),
            _full_spec(HID, OUT_CH),
        ],
        out_specs=_row_spec(OUT_CH),
        out_shape=jax.ShapeDtypeStruct((N_NODES, OUT_CH), jnp.float32),
    )(agg2a, agg2b, ua, ub, dis, b2.reshape(1, OUT_CH), W2)

    return out


# speedup vs baseline: 1.1630x; 1.1630x over previous
import functools

import jax
import jax.numpy as jnp
from jax import lax
from jax.experimental import pallas as pl
from jax.experimental.pallas import tpu as pltpu
from jax.experimental.pallas import tpu_sc as plsc

N_NODES = 10000
N_EDGES = 320000
IN_CH = 128
HID = 64
OUT_CH = 128

_INFO = plsc.get_sparse_core_info()
NC = _INFO.num_cores
NS = _INFO.num_subcores
NW = NC * NS
EPW = N_EDGES // NW
K = 200
NCH = EPW // K
NBUF = 5
NRND = (NCH - 1) // NBUF
NPAD = 10240
SPT = NPAD // NS
DEG_C = 8

_SC_PARAMS = pltpu.CompilerParams(use_tc_tiling_on_sc=False)


def _sc_mesh():
    return plsc.VectorSubcoreMesh(core_axis_name="c", subcore_axis_name="s")


def _make_deg_kernel():
    @functools.partial(
        pl.kernel,
        mesh=_sc_mesh(),
        compiler_params=_SC_PARAMS,
        out_type=jax.ShapeDtypeStruct((NC, NPAD, DEG_C), jnp.float32),
        scratch_types=[
            pltpu.VMEM((EPW,), jnp.int32),
            pltpu.VMEM((K, DEG_C), jnp.float32),
            pltpu.VMEM_SHARED((NPAD, DEG_C), jnp.float32),
            pltpu.SemaphoreType.DMA,
        ],
    )
    def deg_kernel(ei_hbm, ones_hbm, zeros_hbm, out_hbm, dst_v, ones_v, acc_sh, sem):
        cid = lax.axis_index("c")
        sid = lax.axis_index("s")
        wid = sid * NC + cid

        pltpu.sync_copy(zeros_hbm, acc_sh.at[pl.ds(sid * SPT, SPT)])
        pltpu.sync_copy(ei_hbm.at[1, pl.ds(wid * EPW, EPW)], dst_v)
        pltpu.sync_copy(ones_hbm, ones_v)
        plsc.subcore_barrier()

        def body(j, carry):
            for t in range(8):
                i = 8 * j + t
                pltpu.async_copy(
                    ones_v, acc_sh.at[dst_v.at[pl.ds(i * K, K)]], sem, add=True
                )
            for t in range(8):
                pltpu.make_async_copy(
                    ones_v, acc_sh.at[dst_v.at[pl.ds(0, K)]], sem
                ).wait()
            return carry

        lax.fori_loop(0, NCH // 8, body, 0)
        for t in range(NCH % 8):
            i = (NCH // 8) * 8 + t
            pltpu.async_copy(
                ones_v, acc_sh.at[dst_v.at[pl.ds(i * K, K)]], sem, add=True
            )
        for t in range(NCH % 8):
            pltpu.make_async_copy(ones_v, acc_sh.at[dst_v.at[pl.ds(0, K)]], sem).wait()

        plsc.subcore_barrier()
        pltpu.sync_copy(
            acc_sh.at[pl.ds(sid * SPT, SPT)],
            out_hbm.at[cid, pl.ds(sid * SPT, SPT)],
        )

    return deg_kernel


def _make_agg_kernel(C):
    @functools.partial(
        pl.kernel,
        mesh=_sc_mesh(),
        compiler_params=_SC_PARAMS,
        out_type=jax.ShapeDtypeStruct((NC, NPAD, C), jnp.float32),
        scratch_types=[
            pltpu.VMEM((EPW,), jnp.int32),
            pltpu.VMEM((EPW,), jnp.int32),
            [pltpu.VMEM((K, C), jnp.float32)] * NBUF,
            [pltpu.SemaphoreType.DMA] * NBUF,
            [pltpu.SemaphoreType.DMA] * NBUF,
            pltpu.VMEM_SHARED((NPAD, C), jnp.float32),
        ],
    )
    def agg_kernel(ei_hbm, g_hbm, zeros_hbm, out_hbm,
                   src_v, dst_v, bufs, gsems, ssems, acc_sh):
        cid = lax.axis_index("c")
        sid = lax.axis_index("s")
        wid = sid * NC + cid

        pltpu.sync_copy(zeros_hbm, acc_sh.at[pl.ds(sid * SPT, SPT)])
        pltpu.sync_copy(ei_hbm.at[0, pl.ds(wid * EPW, EPW)], src_v)
        pltpu.sync_copy(ei_hbm.at[1, pl.ds(wid * EPW, EPW)], dst_v)
        plsc.subcore_barrier()

        def gather(i, b):
            return pltpu.async_copy(
                g_hbm.at[src_v.at[pl.ds(i * K, K)]], bufs[b], gsems[b]
            )

        def scatter(i, b):
            return pltpu.async_copy(
                bufs[b], acc_sh.at[dst_v.at[pl.ds(i * K, K)]], ssems[b], add=True
            )

        def wait_gather(b):
            pltpu.make_async_copy(g_hbm.at[src_v.at[pl.ds(0, K)]], bufs[b],
                                  gsems[b]).wait()

        def wait_scatter(b):
            pltpu.make_async_copy(bufs[b], acc_sh.at[dst_v.at[pl.ds(0, K)]],
                                  ssems[b]).wait()

        for b in range(NBUF):
            gather(b, b)

        def body(j, carry):
            i0 = NBUF * j
            for b in range(NBUF):
                wait_gather(b)
                scatter(i0 + b, b)
            for b in range(NBUF):
                wait_scatter(b)
                nxt = i0 + NBUF + b

                @pl.when(nxt < NCH)
                def _():
                    gather(nxt, b)

            return carry

        lax.fori_loop(0, NRND, body, 0)
        for t in range(NCH - NBUF * NRND):
            wait_gather(t)
            scatter(NBUF * NRND + t, t)
        for t in range(NCH - NBUF * NRND):
            wait_scatter(t)

        plsc.subcore_barrier()
        pltpu.sync_copy(
            acc_sh.at[pl.ds(sid * SPT, SPT)],
            out_hbm.at[cid, pl.ds(sid * SPT, SPT)],
        )

    return agg_kernel


_deg_kernel = _make_deg_kernel()
_agg_kernel = _make_agg_kernel(HID)


_RB = 2000
_GRID = N_NODES // _RB


def _tc1a_body(x_ref, w1_ref, h_ref):
    h_ref[...] = jnp.dot(
        x_ref[...], w1_ref[...], preferred_element_type=jnp.float32
    )


def _tc1b_body(h_ref, d_ref, g1_ref, dis_ref):
    deg = d_ref[0] + d_ref[1] + 1.0
    dis = lax.rsqrt(deg)
    g1_ref[...] = h_ref[...] * dis
    dis_ref[...] = jnp.broadcast_to(dis, (_RB, 8))


def _tc2_body(p_ref, g1_ref, dis_ref, b1_ref, u_ref):
    dis = dis_ref[:, 0:1]
    s = p_ref[0] + p_ref[1] + g1_ref[...]
    u_ref[...] = dis * jnp.maximum(dis * s + b1_ref[...], 0.0)


def _tc3_body(q_ref, u_ref, dis_ref, b2_ref, w2_ref, out_ref):
    dis = dis_ref[:, 0:1]
    s = dis * (q_ref[0] + q_ref[1] + u_ref[...])
    out_ref[...] = (
        jnp.dot(s, w2_ref[...], preferred_element_type=jnp.float32) + b2_ref[...]
    )


def _row_spec(c):
    return pl.BlockSpec((_RB, c), lambda i: (i, 0))


def _part_spec(c):
    return pl.BlockSpec((NC, _RB, c), lambda i: (0, i, 0))


def _full_spec(r, c):
    return pl.BlockSpec((r, c), lambda i: (0, 0))


def kernel(x, edge_index, W1, b1, W2, b2):
    ei = edge_index.astype(jnp.int32)

    ones_deg = jnp.ones((K, DEG_C), jnp.float32)
    zeros_deg = jnp.zeros((SPT, DEG_C), jnp.float32)
    zeros_h = jnp.zeros((SPT, HID), jnp.float32)

    deg_parts = _deg_kernel(ei, ones_deg, zeros_deg)[:, :, 0:1]

    h1 = pl.pallas_call(
        _tc1a_body,
        grid=(_GRID,),
        in_specs=[_row_spec(IN_CH), _full_spec(IN_CH, HID)],
        out_specs=_row_spec(HID),
        out_shape=jax.ShapeDtypeStruct((N_NODES, HID), jnp.float32),
    )(x, W1)

    g1, dis = pl.pallas_call(
        _tc1b_body,
        grid=(_GRID,),
        in_specs=[_row_spec(HID), _part_spec(1)],
        out_specs=[_row_spec(HID), _row_spec(8)],
        out_shape=[
            jax.ShapeDtypeStruct((N_NODES, HID), jnp.float32),
            jax.ShapeDtypeStruct((N_NODES, 8), jnp.float32),
        ],
    )(h1, deg_parts)

    agg1 = _agg_kernel(ei, g1, zeros_h)

    u = pl.pallas_call(
        _tc2_body,
        grid=(_GRID,),
        in_specs=[
            _part_spec(HID),
            _row_spec(HID),
            _row_spec(8),
            _full_spec(1, HID),
        ],
        out_specs=_row_spec(HID),
        out_shape=jax.ShapeDtypeStruct((N_NODES, HID), jnp.float32),
    )(agg1, g1, dis, b1.reshape(1, HID))

    agg2 = _agg_kernel(ei, u, zeros_h)

    out = pl.pallas_call(
        _tc3_body,
        grid=(_GRID,),
        in_specs=[
            _part_spec(HID),
            _row_spec(HID),
            _row_spec(8),
            _full_spec(1, OUT_CH),
            _full_spec(HID, OUT_CH),
        ],
        out_specs=_row_spec(OUT_CH),
        out_shape=jax.ShapeDtypeStruct((N_NODES, OUT_CH), jnp.float32),
    )(agg2, u, dis, b2.reshape(1, OUT_CH), W2)

    return out
